# Initial kernel scaffold; baseline (speedup 1.0000x reference)
import jax
import jax.numpy as jnp
from jax.experimental import pallas as pl


def kernel(instruction, table):
    raise NotImplementedError("write your pallas kernel here")


# speedup vs baseline: 1.5163x; 1.5163x over previous
import functools

import jax
import jax.numpy as jnp
from jax import lax
from jax.experimental import pallas as pl
from jax.experimental.pallas import tpu as pltpu
from jax.experimental.pallas import tpu_sc as plsc

LANES = 16
SLAB = 128
SLABS_PER_CHUNK = 8
CHUNK = SLAB * SLABS_PER_CHUNK
GROUPS = SLAB // LANES
BIG = 0x7FFFFFFF


def _build_lookup(bs: int, embed: int, num_workers: int):
    rows_per_worker = bs // num_workers
    n_chunks = rows_per_worker // CHUNK
    slab_rows_per_worker = rows_per_worker // SLAB

    mesh = plsc.VectorSubcoreMesh(core_axis_name="c", subcore_axis_name="s")

    @functools.partial(
        pl.kernel,
        mesh=mesh,
        compiler_params=pltpu.CompilerParams(use_tc_tiling_on_sc=False),
        out_type=jax.ShapeDtypeStruct((bs, embed), jnp.float32),
        scratch_types=[
            pltpu.VMEM((SLABS_PER_CHUNK, SLAB), jnp.int32),
            pltpu.VMEM((CHUNK, embed), jnp.float32),
            pltpu.VMEM((SLABS_PER_CHUNK, SLAB), jnp.int32),
            pltpu.VMEM((SLAB, embed), jnp.float32),
            pltpu.SemaphoreType.DMA,
        ],
    )
    def lookup(table_hbm, idx_hbm, out_hbm, idx_v, rows_v, padpos_v, zero_v, sem):
        n_cores = lax.axis_size("c")
        wid = lax.axis_index("s") * n_cores + lax.axis_index("c")
        row_base = wid * rows_per_worker
        slab_base = wid * slab_rows_per_worker
        iot = lax.iota(jnp.int32, LANES)

        def zfill(i, c):
            zero_v[i, pl.ds(0, LANES)] = jnp.zeros((LANES,), jnp.float32)
            zero_v[i, pl.ds(LANES, LANES)] = jnp.zeros((LANES,), jnp.float32)
            return c

        lax.fori_loop(0, SLAB, zfill, 0)

        def do_chunk(g, carry):
            chunk_row = row_base + g * CHUNK
            pltpu.sync_copy(
                idx_hbm.at[pl.ds(slab_base + g * SLABS_PER_CHUNK, SLABS_PER_CHUNK)],
                idx_v,
            )
            copies = [
                pltpu.async_copy(
                    table_hbm.at[idx_v.at[j]],
                    rows_v.at[pl.ds(j * SLAB, SLAB)],
                    sem,
                )
                for j in range(SLABS_PER_CHUNK)
            ]
            for cp in copies:
                cp.wait()

            minpos = jnp.full((LANES,), BIG, jnp.int32)
            for j in range(SLABS_PER_CHUNK):
                for k in range(GROUPS):
                    v = idx_v[j, pl.ds(k * LANES, LANES)]
                    gpos = chunk_row + j * SLAB + k * LANES + iot
                    minpos = jnp.minimum(minpos, jnp.where(v == 0, gpos, BIG))
            s = minpos[0]
            for i in range(1, LANES):
                s = jnp.minimum(s, minpos[i])

            pltpu.sync_copy(rows_v, out_hbm.at[pl.ds(chunk_row, CHUNK)])

            @pl.when(s != BIG)
            def _():
                for j in range(SLABS_PER_CHUNK):
                    for k in range(GROUPS):
                        v = idx_v[j, pl.ds(k * LANES, LANES)]
                        gpos = chunk_row + j * SLAB + k * LANES + iot
                        padpos_v[j, pl.ds(k * LANES, LANES)] = jnp.where(
                            v == 0, gpos, s)
                zcopies = [
                    pltpu.async_copy(zero_v, out_hbm.at[padpos_v.at[j]], sem)
                    for j in range(SLABS_PER_CHUNK)
                ]
                for cp in zcopies:
                    cp.wait()

            return carry

        lax.fori_loop(0, n_chunks, do_chunk, 0)

    return lookup


def kernel(instruction, table):
    b, s = instruction.shape
    vocab, embed = table.shape
    bs = b * s
    idx = instruction.astype(jnp.int32).reshape(bs // SLAB, SLAB)
    info = plsc.get_sparse_core_info()
    num_workers = info.num_cores * info.num_subcores
    out = _build_lookup(bs, embed, num_workers)(table, idx)
    return out.reshape(b, s, embed)
